# baseline (speedup 1.0000x reference)
import functools

import jax
import jax.numpy as jnp
from jax import lax
from jax.experimental import pallas as pl
from jax.experimental.pallas import tpu as pltpu
from jax.experimental.pallas import tpu_sc as plsc

NC = 2
NS = 16
NW = NC * NS
CH = 80

_MESH = plsc.VectorSubcoreMesh(core_axis_name="c", subcore_axis_name="s")


@functools.lru_cache(maxsize=None)
def _degree_kernel(E, NPAD, SEC, SCH):
    CP = NPAD // NS
    SP = ((CP + 15) // 16) * 16

    @functools.partial(
        pl.kernel,
        out_type=jax.ShapeDtypeStruct((NC * 4 * NPAD,), jnp.float32),
        mesh=_MESH,
        scratch_types=[
            pltpu.VMEM_SHARED((NPAD,), jnp.float32),
            pltpu.VMEM_SHARED((NPAD,), jnp.float32),
            pltpu.VMEM_SHARED((NPAD,), jnp.float32),
            pltpu.VMEM_SHARED((NPAD,), jnp.float32),
            pltpu.VMEM((SCH, CH), jnp.int32),
            pltpu.VMEM((CH,), jnp.float32),
            pltpu.VMEM((SP,), jnp.float32),
            pltpu.SemaphoreType.DMA,
        ],
    )
    def body(s0, d0, s1, d1, out, g0, g1, g2, g3, idx, ones, stage, sem):
        c = lax.axis_index("c")
        s = lax.axis_index("s")
        wid = s * NC + c
        hists = (g0, g1, g2, g3)
        z16 = jnp.zeros((16,), jnp.float32)
        for j in range(SP // 16):
            stage[pl.ds(j * 16, 16)] = z16
        o16 = jnp.ones((16,), jnp.float32)
        for j in range(CH // 16):
            ones[pl.ds(j * 16, 16)] = o16
        for g in hists:
            pltpu.sync_copy(stage.at[pl.ds(0, CP)], g.at[pl.ds(s * CP, CP)])
        plsc.subcore_barrier()
        for g, arr in zip(hists, (s0, d0, s1, d1)):
            def section(sec, _, g=g, arr=arr):
                pltpu.sync_copy(arr.at[wid].at[sec], idx)

                def fire(k, _):
                    pltpu.async_copy(ones, g.at[idx.at[k]], sem, add=True)
                    return _

                lax.fori_loop(0, SCH, fire, None)

                def drain(k, _):
                    pltpu.make_async_copy(ones, g.at[idx.at[0]], sem).wait()
                    return _

                lax.fori_loop(0, SCH, drain, None)
                return _

            lax.fori_loop(0, SEC, section, None)
        plsc.subcore_barrier()
        for a, g in enumerate(hists):
            pltpu.sync_copy(g.at[pl.ds(s * CP, CP)], stage.at[pl.ds(0, CP)])
            pltpu.sync_copy(stage.at[pl.ds(0, CP)],
                            out.at[pl.ds((c * 4 + a) * NPAD + s * CP, CP)])

    return body


@functools.lru_cache(maxsize=None)
def _aggregate_kernel(N, D, E, NR, SEC, SCH):
    assert SCH >= 6 and (SCH - 5) % 4 == 0
    NQUAD = (SCH - 5) // 4
    RP = NR // NS

    @functools.partial(
        pl.kernel,
        out_type=jax.ShapeDtypeStruct((NC, NR, D), jnp.float32),
        mesh=_MESH,
        scratch_types=[
            pltpu.VMEM_SHARED((NR, D), jnp.float32),
            pltpu.VMEM((SCH, CH), jnp.int32),
            pltpu.VMEM((SCH, CH), jnp.int32),
            pltpu.VMEM((CH, D), jnp.float32),
            pltpu.VMEM((CH, D), jnp.float32),
            pltpu.VMEM((CH, D), jnp.float32),
            pltpu.VMEM((CH, D), jnp.float32),
            pltpu.SemaphoreType.DMA,
            pltpu.SemaphoreType.DMA,
            pltpu.SemaphoreType.DMA,
            pltpu.SemaphoreType.DMA,
            pltpu.SemaphoreType.DMA,
            pltpu.SemaphoreType.DMA,
            pltpu.SemaphoreType.DMA,
            pltpu.SemaphoreType.DMA,
        ],
    )
    def body(y, src, dst, out, acc, sidx, didx,
             rows0, rows1, rows2, rows3,
             ga, gb, gc, gd, sa, sb, sc_, sd):
        c = lax.axis_index("c")
        s = lax.axis_index("s")
        wid = s * NC + c
        R = (rows0, rows1, rows2, rows3)
        G = (ga, gb, gc, gd)
        S = (sa, sb, sc_, sd)

        z16 = jnp.zeros((16,), jnp.float32)
        for r in range(8):
            for j in range(D // 16):
                rows0[r, pl.ds(j * 16, 16)] = z16

        def zfire(k, _):
            pltpu.async_copy(rows0.at[pl.ds(0, 8)],
                             acc.at[pl.ds(s * RP + k * 8, 8)], sa)
            return _

        lax.fori_loop(0, RP // 8, zfire, None)

        def zdrain(k, _):
            pltpu.make_async_copy(rows0.at[pl.ds(0, 8)],
                                  acc.at[pl.ds(s * RP, 8)], sa).wait()
            return _

        lax.fori_loop(0, RP // 8, zdrain, None)
        plsc.subcore_barrier()

        def gath(k, b):
            pltpu.async_copy(y.at[sidx.at[k]], R[b], G[b])

        def gwait(b):
            pltpu.make_async_copy(y.at[sidx.at[0]], R[b], G[b]).wait()

        def scat(k, b):
            pltpu.async_copy(R[b], acc.at[didx.at[k]], S[b], add=True)

        def swait(b):
            pltpu.make_async_copy(R[b], acc.at[didx.at[0]], S[b]).wait()

        def section(sec, _):
            pltpu.sync_copy(src.at[wid].at[sec], sidx)
            pltpu.sync_copy(dst.at[wid].at[sec], didx)
            gath(0, 0)
            gath(1, 1)
            gwait(0)
            scat(0, 0)
            gath(2, 2)
            gwait(1)
            scat(1, 1)
            gath(3, 3)
            gwait(2)
            scat(2, 2)

            def quad(j, _):
                base = 4 + 4 * j
                for t in range(4):
                    k = base + t
                    swait(t)
                    gath(k, t)
                    bp = (t + 3) % 4
                    gwait(bp)
                    scat(k - 1, bp)
                return _

            lax.fori_loop(0, NQUAD, quad, None)
            last = SCH - 1
            swait(last % 4)
            gath(last, last % 4)
            gwait((last + 3) % 4)
            scat(last - 1, (last + 3) % 4)
            gwait(last % 4)
            scat(last, last % 4)
            for b in range(4):
                swait(b)
            return _

        lax.fori_loop(0, SEC, section, None)
        plsc.subcore_barrier()

        nfull, rem = divmod(RP, CH)
        sizes = [CH] * nfull + ([rem] if rem else [])
        pend = [None, None, None, None]
        for m, sz in enumerate(sizes):
            b = m % 4
            if pend[b] is not None:
                po, psz = pend[b]
                pltpu.make_async_copy(
                    R[b].at[pl.ds(0, psz)],
                    out.at[c].at[pl.ds(s * RP + po, psz)], S[b]).wait()
            off = m * CH
            pltpu.sync_copy(acc.at[pl.ds(s * RP + off, sz)],
                            R[b].at[pl.ds(0, sz)])
            pltpu.async_copy(R[b].at[pl.ds(0, sz)],
                             out.at[c].at[pl.ds(s * RP + off, sz)], S[b])
            pend[b] = (off, sz)
        for b in range(4):
            if pend[b] is not None:
                po, psz = pend[b]
                pltpu.make_async_copy(
                    R[b].at[pl.ds(0, psz)],
                    out.at[c].at[pl.ds(s * RP + po, psz)], S[b]).wait()

    return body


def _tc_pre(N):
    def body(x_ref, degp_ref, w_ref, y_ref):
        od = (degp_ref[0, 0, :] + degp_ref[1, 0, :])[:N]
        norm = lax.rsqrt(jnp.maximum(od, 1.0))
        h = x_ref[...] * norm[:, None]
        y_ref[...] = jnp.dot(h, w_ref[...], preferred_element_type=jnp.float32)
    return body


def _tc_mid(N):
    def body(sp_ref, degp_ref, b_ref, w_ref, y_ref):
        agg = sp_ref[0, :N] + sp_ref[1, :N]
        ind = (degp_ref[0, 1, :] + degp_ref[1, 1, :])[:N]
        od = (degp_ref[0, 2, :] + degp_ref[1, 2, :])[:N]
        h = jnp.maximum(agg * lax.rsqrt(jnp.maximum(ind, 1.0))[:, None]
                        + b_ref[...][None, :], 0.0)
        h = h * lax.rsqrt(jnp.maximum(od, 1.0))[:, None]
        y_ref[...] = jnp.dot(h, w_ref[...], preferred_element_type=jnp.float32)
    return body


def _tc_post(N):
    def body(sp_ref, degp_ref, b_ref, o_ref):
        agg = sp_ref[0, :N] + sp_ref[1, :N]
        ind = (degp_ref[0, 3, :] + degp_ref[1, 3, :])[:N]
        o_ref[...] = jnp.maximum(agg * lax.rsqrt(jnp.maximum(ind, 1.0))[:, None]
                                 + b_ref[...][None, :], 0.0)
    return body


def kernel(x, edge_index0, edge_index1, W0, b0, W1, b1):
    N, D = x.shape
    E = edge_index0.shape[1]
    NPAD = ((N + 127) // 128) * 128
    NCHUNK = E // NW // CH
    SCH = 25 if NCHUNK % 25 == 0 else NCHUNK
    SEC = NCHUNK // SCH

    s0 = edge_index0[0].reshape(NW, SEC, SCH, CH)
    d0 = edge_index0[1].reshape(NW, SEC, SCH, CH)
    s1 = edge_index1[0].reshape(NW, SEC, SCH, CH)
    d1 = edge_index1[1].reshape(NW, SEC, SCH, CH)

    degp = _degree_kernel(E, NPAD, SEC, SCH)(s0, d0, s1, d1)
    degp = degp.reshape(NC, 4, NPAD)

    f32 = jnp.float32
    y0 = pl.pallas_call(
        _tc_pre(N), out_shape=jax.ShapeDtypeStruct((N, D), f32))(x, degp, W0)
    sp0 = _aggregate_kernel(N, D, E, NPAD, SEC, SCH)(y0, s0, d0)
    y1 = pl.pallas_call(
        _tc_mid(N), out_shape=jax.ShapeDtypeStruct((N, D), f32))(
            sp0, degp, b0, W1)
    sp1 = _aggregate_kernel(N, D, E, NPAD, SEC, SCH)(y1, s1, d1)
    out = pl.pallas_call(
        _tc_post(N), out_shape=jax.ShapeDtypeStruct((N, D), f32))(
            sp1, degp, b1)
    return out


# speedup vs baseline: 1.0848x; 1.0848x over previous
import functools

import jax
import jax.numpy as jnp
from jax import lax
from jax.experimental import pallas as pl
from jax.experimental.pallas import tpu as pltpu
from jax.experimental.pallas import tpu_sc as plsc

NC = 2
NS = 16
NW = NC * NS
CH = 80

_MESH = plsc.VectorSubcoreMesh(core_axis_name="c", subcore_axis_name="s")


@functools.lru_cache(maxsize=None)
def _degree_kernel(E, NPAD, SEC, SCH):
    CP = NPAD // NS
    SP = ((CP + 15) // 16) * 16

    @functools.partial(
        pl.kernel,
        out_type=jax.ShapeDtypeStruct((NC * 4 * NPAD,), jnp.float32),
        mesh=_MESH,
        scratch_types=[
            pltpu.VMEM_SHARED((NPAD,), jnp.float32),
            pltpu.VMEM_SHARED((NPAD,), jnp.float32),
            pltpu.VMEM_SHARED((NPAD,), jnp.float32),
            pltpu.VMEM_SHARED((NPAD,), jnp.float32),
            pltpu.VMEM((SCH, CH), jnp.int32),
            pltpu.VMEM((CH,), jnp.float32),
            pltpu.VMEM((SP,), jnp.float32),
            pltpu.SemaphoreType.DMA,
        ],
    )
    def body(s0, d0, s1, d1, out, g0, g1, g2, g3, idx, ones, stage, sem):
        c = lax.axis_index("c")
        s = lax.axis_index("s")
        wid = s * NC + c
        hists = (g0, g1, g2, g3)
        z16 = jnp.zeros((16,), jnp.float32)
        for j in range(SP // 16):
            stage[pl.ds(j * 16, 16)] = z16
        o16 = jnp.ones((16,), jnp.float32)
        for j in range(CH // 16):
            ones[pl.ds(j * 16, 16)] = o16
        for g in hists:
            pltpu.sync_copy(stage.at[pl.ds(0, CP)], g.at[pl.ds(s * CP, CP)])
        plsc.subcore_barrier()
        for g, arr in zip(hists, (s0, d0, s1, d1)):
            def section(sec, _, g=g, arr=arr):
                pltpu.sync_copy(arr.at[wid].at[sec], idx)

                def fire(k, _):
                    pltpu.async_copy(ones, g.at[idx.at[k]], sem, add=True)
                    return _

                lax.fori_loop(0, SCH, fire, None)

                def drain(k, _):
                    pltpu.make_async_copy(ones, g.at[idx.at[0]], sem).wait()
                    return _

                lax.fori_loop(0, SCH, drain, None)
                return _

            lax.fori_loop(0, SEC, section, None)
        plsc.subcore_barrier()
        for a, g in enumerate(hists):
            pltpu.sync_copy(g.at[pl.ds(s * CP, CP)], stage.at[pl.ds(0, CP)])
            pltpu.sync_copy(stage.at[pl.ds(0, CP)],
                            out.at[pl.ds((c * 4 + a) * NPAD + s * CP, CP)])

    return body


@functools.lru_cache(maxsize=None)
def _aggregate_kernel(N, D, E, NR, SEC, SCH):
    assert SCH >= 5 and (SCH - 4) % 3 == 0
    NTRIP = (SCH - 4) // 3
    RP = NR // NS

    @functools.partial(
        pl.kernel,
        out_type=jax.ShapeDtypeStruct((NC, NR, D), jnp.float32),
        mesh=_MESH,
        scratch_types=[
            pltpu.VMEM_SHARED((NR, D), jnp.float32),
            pltpu.VMEM((SCH, CH), jnp.int32),
            pltpu.VMEM((SCH, CH), jnp.int32),
            pltpu.VMEM((CH, D), jnp.float32),
            pltpu.VMEM((CH, D), jnp.float32),
            pltpu.VMEM((CH, D), jnp.float32),
            pltpu.SemaphoreType.DMA,
            pltpu.SemaphoreType.DMA,
            pltpu.SemaphoreType.DMA,
            pltpu.SemaphoreType.DMA,
            pltpu.SemaphoreType.DMA,
            pltpu.SemaphoreType.DMA,
        ],
    )
    def body(y, src, dst, out, acc, sidx, didx,
             rows0, rows1, rows2, ga, gb, gc, sa, sb, sc_):
        c = lax.axis_index("c")
        s = lax.axis_index("s")
        wid = s * NC + c
        R = (rows0, rows1, rows2)
        G = (ga, gb, gc)
        S = (sa, sb, sc_)

        z16 = jnp.zeros((16,), jnp.float32)
        for r in range(8):
            for j in range(D // 16):
                rows0[r, pl.ds(j * 16, 16)] = z16

        def zfire(k, _):
            pltpu.async_copy(rows0.at[pl.ds(0, 8)],
                             acc.at[pl.ds(s * RP + k * 8, 8)], sa)
            return _

        lax.fori_loop(0, RP // 8, zfire, None)

        def zdrain(k, _):
            pltpu.make_async_copy(rows0.at[pl.ds(0, 8)],
                                  acc.at[pl.ds(s * RP, 8)], sa).wait()
            return _

        lax.fori_loop(0, RP // 8, zdrain, None)
        plsc.subcore_barrier()

        def gath(k, b):
            pltpu.async_copy(y.at[sidx.at[k]], R[b], G[b])

        def gwait(b):
            pltpu.make_async_copy(y.at[sidx.at[0]], R[b], G[b]).wait()

        def scat(k, b):
            pltpu.async_copy(R[b], acc.at[didx.at[k]], S[b], add=True)

        def swait(b):
            pltpu.make_async_copy(R[b], acc.at[didx.at[0]], S[b]).wait()

        def section(sec, _):
            pltpu.sync_copy(src.at[wid].at[sec], sidx)
            pltpu.sync_copy(dst.at[wid].at[sec], didx)
            gath(0, 0)
            gath(1, 1)
            gwait(0)
            scat(0, 0)
            gath(2, 2)
            gwait(1)
            scat(1, 1)
            def trip(j, _):
                base = 3 + 3 * j
                for t in range(3):
                    k = base + t
                    swait(t)
                    gath(k, t)
                    bp = (t + 2) % 3
                    gwait(bp)
                    scat(k - 1, bp)
                return _

            lax.fori_loop(0, NTRIP, trip, None)
            last = SCH - 1
            swait(last % 3)
            gath(last, last % 3)
            gwait((last + 2) % 3)
            scat(last - 1, (last + 2) % 3)
            gwait(last % 3)
            scat(last, last % 3)
            for b in range(3):
                swait(b)
            return _

        lax.fori_loop(0, SEC, section, None)
        plsc.subcore_barrier()

        nfull, rem = divmod(RP, CH)
        sizes = [CH] * nfull + ([rem] if rem else [])
        pend = [None, None, None]
        for m, sz in enumerate(sizes):
            b = m % 3
            if pend[b] is not None:
                po, psz = pend[b]
                pltpu.make_async_copy(
                    R[b].at[pl.ds(0, psz)],
                    out.at[c].at[pl.ds(s * RP + po, psz)], S[b]).wait()
            off = m * CH
            pltpu.sync_copy(acc.at[pl.ds(s * RP + off, sz)],
                            R[b].at[pl.ds(0, sz)])
            pltpu.async_copy(R[b].at[pl.ds(0, sz)],
                             out.at[c].at[pl.ds(s * RP + off, sz)], S[b])
            pend[b] = (off, sz)
        for b in range(3):
            if pend[b] is not None:
                po, psz = pend[b]
                pltpu.make_async_copy(
                    R[b].at[pl.ds(0, psz)],
                    out.at[c].at[pl.ds(s * RP + po, psz)], S[b]).wait()

    return body


def _tc_pre(N):
    def body(x_ref, degp_ref, w_ref, y_ref):
        od = (degp_ref[0, 0, :] + degp_ref[1, 0, :])[:N]
        norm = lax.rsqrt(jnp.maximum(od, 1.0))
        h = x_ref[...] * norm[:, None]
        y_ref[...] = jnp.dot(h, w_ref[...], preferred_element_type=jnp.float32)
    return body


def _tc_mid(N):
    def body(sp_ref, degp_ref, b_ref, w_ref, y_ref):
        agg = sp_ref[0, :N] + sp_ref[1, :N]
        ind = (degp_ref[0, 1, :] + degp_ref[1, 1, :])[:N]
        od = (degp_ref[0, 2, :] + degp_ref[1, 2, :])[:N]
        h = jnp.maximum(agg * lax.rsqrt(jnp.maximum(ind, 1.0))[:, None]
                        + b_ref[...][None, :], 0.0)
        h = h * lax.rsqrt(jnp.maximum(od, 1.0))[:, None]
        y_ref[...] = jnp.dot(h, w_ref[...], preferred_element_type=jnp.float32)
    return body


def _tc_post(N):
    def body(sp_ref, degp_ref, b_ref, o_ref):
        agg = sp_ref[0, :N] + sp_ref[1, :N]
        ind = (degp_ref[0, 3, :] + degp_ref[1, 3, :])[:N]
        o_ref[...] = jnp.maximum(agg * lax.rsqrt(jnp.maximum(ind, 1.0))[:, None]
                                 + b_ref[...][None, :], 0.0)
    return body


def kernel(x, edge_index0, edge_index1, W0, b0, W1, b1):
    N, D = x.shape
    E = edge_index0.shape[1]
    NPAD = ((N + 127) // 128) * 128
    NCHUNK = E // NW // CH
    SCH = 25 if NCHUNK % 25 == 0 else NCHUNK
    SEC = NCHUNK // SCH

    s0 = edge_index0[0].reshape(NW, SEC, SCH, CH)
    d0 = edge_index0[1].reshape(NW, SEC, SCH, CH)
    s1 = edge_index1[0].reshape(NW, SEC, SCH, CH)
    d1 = edge_index1[1].reshape(NW, SEC, SCH, CH)

    degp = _degree_kernel(E, NPAD, SEC, SCH)(s0, d0, s1, d1)
    degp = degp.reshape(NC, 4, NPAD)

    f32 = jnp.float32
    y0 = pl.pallas_call(
        _tc_pre(N), out_shape=jax.ShapeDtypeStruct((N, D), f32))(x, degp, W0)
    sp0 = _aggregate_kernel(N, D, E, NPAD, SEC, SCH)(y0, s0, d0)
    y1 = pl.pallas_call(
        _tc_mid(N), out_shape=jax.ShapeDtypeStruct((N, D), f32))(
            sp0, degp, b0, W1)
    sp1 = _aggregate_kernel(N, D, E, NPAD, SEC, SCH)(y1, s1, d1)
    out = pl.pallas_call(
        _tc_post(N), out_shape=jax.ShapeDtypeStruct((N, D), f32))(
            sp1, degp, b1)
    return out


# speedup vs baseline: 1.1145x; 1.0274x over previous
import functools

import jax
import jax.numpy as jnp
from jax import lax
from jax.experimental import pallas as pl
from jax.experimental.pallas import tpu as pltpu
from jax.experimental.pallas import tpu_sc as plsc

NC = 2
NS = 16
NW = NC * NS
CH = 80

_MESH = plsc.VectorSubcoreMesh(core_axis_name="c", subcore_axis_name="s")


@functools.lru_cache(maxsize=None)
def _degree_kernel(E, NPAD, SEC, SCH):
    CP = NPAD // NS
    SP = ((CP + 15) // 16) * 16

    @functools.partial(
        pl.kernel,
        out_type=jax.ShapeDtypeStruct((NC * 4 * NPAD,), jnp.float32),
        mesh=_MESH,
        scratch_types=[
            pltpu.VMEM_SHARED((NPAD,), jnp.float32),
            pltpu.VMEM_SHARED((NPAD,), jnp.float32),
            pltpu.VMEM_SHARED((NPAD,), jnp.float32),
            pltpu.VMEM_SHARED((NPAD,), jnp.float32),
            pltpu.VMEM((SEC, SCH, CH), jnp.int32),
            pltpu.VMEM((SEC, SCH, CH), jnp.int32),
            pltpu.VMEM((SEC, SCH, CH), jnp.int32),
            pltpu.VMEM((SEC, SCH, CH), jnp.int32),
            pltpu.VMEM((CH,), jnp.float32),
            pltpu.VMEM((SP,), jnp.float32),
            pltpu.SemaphoreType.DMA,
        ],
    )
    def body(s0, d0, s1, d1, out, g0, g1, g2, g3,
             i0, i1, i2, i3, ones, stage, sem):
        c = lax.axis_index("c")
        s = lax.axis_index("s")
        wid = s * NC + c
        hists = (g0, g1, g2, g3)
        ibufs = (i0, i1, i2, i3)
        z16 = jnp.zeros((16,), jnp.float32)
        for j in range(SP // 16):
            stage[pl.ds(j * 16, 16)] = z16
        o16 = jnp.ones((16,), jnp.float32)
        for j in range(CH // 16):
            ones[pl.ds(j * 16, 16)] = o16
        for g in hists:
            pltpu.sync_copy(stage.at[pl.ds(0, CP)], g.at[pl.ds(s * CP, CP)])
        for arr, ib in zip((s0, d0, s1, d1), ibufs):
            pltpu.sync_copy(arr.at[wid], ib)
        plsc.subcore_barrier()
        for g, ib in zip(hists, ibufs):
            def osec(sec, _, g=g, ib=ib):
                def fire(k, _):
                    pltpu.async_copy(ones, g.at[ib.at[sec].at[k]],
                                     sem, add=True)
                    return _

                lax.fori_loop(0, SCH, fire, None)
                return _

            lax.fori_loop(0, SEC, osec, None)

        def drain(k, _):
            pltpu.make_async_copy(ones, g0.at[i0.at[0].at[0]], sem).wait()
            return _

        lax.fori_loop(0, 4 * SEC * SCH, drain, None)
        plsc.subcore_barrier()
        for a, g in enumerate(hists):
            pltpu.sync_copy(g.at[pl.ds(s * CP, CP)], stage.at[pl.ds(0, CP)])
            pltpu.sync_copy(stage.at[pl.ds(0, CP)],
                            out.at[pl.ds((c * 4 + a) * NPAD + s * CP, CP)])

    return body


@functools.lru_cache(maxsize=None)
def _aggregate_kernel(N, D, E, NR, SEC, SCH):
    assert SCH >= 5 and (SCH - 4) % 3 == 0
    NTRIP = (SCH - 4) // 3
    RP = NR // NS

    @functools.partial(
        pl.kernel,
        out_type=jax.ShapeDtypeStruct((NC, NR, D), jnp.float32),
        mesh=_MESH,
        scratch_types=[
            pltpu.VMEM_SHARED((NR, D), jnp.float32),
            pltpu.VMEM((SCH, CH), jnp.int32),
            pltpu.VMEM((SCH, CH), jnp.int32),
            pltpu.VMEM((CH, D), jnp.float32),
            pltpu.VMEM((CH, D), jnp.float32),
            pltpu.VMEM((CH, D), jnp.float32),
            pltpu.SemaphoreType.DMA,
            pltpu.SemaphoreType.DMA,
            pltpu.SemaphoreType.DMA,
            pltpu.SemaphoreType.DMA,
            pltpu.SemaphoreType.DMA,
            pltpu.SemaphoreType.DMA,
        ],
    )
    def body(y, src, dst, out, acc, sidx, didx,
             rows0, rows1, rows2, ga, gb, gc, sa, sb, sc_):
        c = lax.axis_index("c")
        s = lax.axis_index("s")
        wid = s * NC + c
        R = (rows0, rows1, rows2)
        G = (ga, gb, gc)
        S = (sa, sb, sc_)

        z16 = jnp.zeros((16,), jnp.float32)
        for r in range(8):
            for j in range(D // 16):
                rows0[r, pl.ds(j * 16, 16)] = z16

        def zfire(k, _):
            pltpu.async_copy(rows0.at[pl.ds(0, 8)],
                             acc.at[pl.ds(s * RP + k * 8, 8)], sa)
            return _

        lax.fori_loop(0, RP // 8, zfire, None)

        def zdrain(k, _):
            pltpu.make_async_copy(rows0.at[pl.ds(0, 8)],
                                  acc.at[pl.ds(s * RP, 8)], sa).wait()
            return _

        lax.fori_loop(0, RP // 8, zdrain, None)
        plsc.subcore_barrier()

        def gath(k, b):
            pltpu.async_copy(y.at[sidx.at[k]], R[b], G[b])

        def gwait(b):
            pltpu.make_async_copy(y.at[sidx.at[0]], R[b], G[b]).wait()

        def scat(k, b):
            pltpu.async_copy(R[b], acc.at[didx.at[k]], S[b], add=True)

        def swait(b):
            pltpu.make_async_copy(R[b], acc.at[didx.at[0]], S[b]).wait()

        def section(sec, _):
            pltpu.sync_copy(src.at[wid].at[sec], sidx)
            pltpu.sync_copy(dst.at[wid].at[sec], didx)
            gath(0, 0)
            gath(1, 1)
            gwait(0)
            scat(0, 0)
            gath(2, 2)
            gwait(1)
            scat(1, 1)
            def trip(j, _):
                base = 3 + 3 * j
                for t in range(3):
                    k = base + t
                    swait(t)
                    gath(k, t)
                    bp = (t + 2) % 3
                    gwait(bp)
                    scat(k - 1, bp)
                return _

            lax.fori_loop(0, NTRIP, trip, None)
            last = SCH - 1
            swait(last % 3)
            gath(last, last % 3)
            gwait((last + 2) % 3)
            scat(last - 1, (last + 2) % 3)
            gwait(last % 3)
            scat(last, last % 3)
            for b in range(3):
                swait(b)
            return _

        lax.fori_loop(0, SEC, section, None)
        plsc.subcore_barrier()

        nfull, rem = divmod(RP, CH)
        sizes = [CH] * nfull + ([rem] if rem else [])
        pend = [None, None, None]
        for m, sz in enumerate(sizes):
            b = m % 3
            if pend[b] is not None:
                po, psz = pend[b]
                pltpu.make_async_copy(
                    R[b].at[pl.ds(0, psz)],
                    out.at[c].at[pl.ds(s * RP + po, psz)], S[b]).wait()
            off = m * CH
            pltpu.sync_copy(acc.at[pl.ds(s * RP + off, sz)],
                            R[b].at[pl.ds(0, sz)])
            pltpu.async_copy(R[b].at[pl.ds(0, sz)],
                             out.at[c].at[pl.ds(s * RP + off, sz)], S[b])
            pend[b] = (off, sz)
        for b in range(3):
            if pend[b] is not None:
                po, psz = pend[b]
                pltpu.make_async_copy(
                    R[b].at[pl.ds(0, psz)],
                    out.at[c].at[pl.ds(s * RP + po, psz)], S[b]).wait()

    return body


def _tc_pre(N):
    def body(x_ref, degp_ref, w_ref, y_ref):
        od = (degp_ref[0, 0, :] + degp_ref[1, 0, :])[:N]
        norm = lax.rsqrt(jnp.maximum(od, 1.0))
        h = x_ref[...] * norm[:, None]
        y_ref[...] = jnp.dot(h, w_ref[...], preferred_element_type=jnp.float32)
    return body


def _tc_mid(N):
    def body(sp_ref, degp_ref, b_ref, w_ref, y_ref):
        agg = sp_ref[0, :N] + sp_ref[1, :N]
        ind = (degp_ref[0, 1, :] + degp_ref[1, 1, :])[:N]
        od = (degp_ref[0, 2, :] + degp_ref[1, 2, :])[:N]
        h = jnp.maximum(agg * lax.rsqrt(jnp.maximum(ind, 1.0))[:, None]
                        + b_ref[...][None, :], 0.0)
        h = h * lax.rsqrt(jnp.maximum(od, 1.0))[:, None]
        y_ref[...] = jnp.dot(h, w_ref[...], preferred_element_type=jnp.float32)
    return body


def _tc_post(N):
    def body(sp_ref, degp_ref, b_ref, o_ref):
        agg = sp_ref[0, :N] + sp_ref[1, :N]
        ind = (degp_ref[0, 3, :] + degp_ref[1, 3, :])[:N]
        o_ref[...] = jnp.maximum(agg * lax.rsqrt(jnp.maximum(ind, 1.0))[:, None]
                                 + b_ref[...][None, :], 0.0)
    return body


def kernel(x, edge_index0, edge_index1, W0, b0, W1, b1):
    N, D = x.shape
    E = edge_index0.shape[1]
    NPAD = ((N + 127) // 128) * 128
    NCHUNK = E // NW // CH
    SCH = 25 if NCHUNK % 25 == 0 else NCHUNK
    SEC = NCHUNK // SCH

    s0 = edge_index0[0].reshape(NW, SEC, SCH, CH)
    d0 = edge_index0[1].reshape(NW, SEC, SCH, CH)
    s1 = edge_index1[0].reshape(NW, SEC, SCH, CH)
    d1 = edge_index1[1].reshape(NW, SEC, SCH, CH)

    degp = _degree_kernel(E, NPAD, SEC, SCH)(s0, d0, s1, d1)
    degp = degp.reshape(NC, 4, NPAD)

    f32 = jnp.float32
    y0 = pl.pallas_call(
        _tc_pre(N), out_shape=jax.ShapeDtypeStruct((N, D), f32))(x, degp, W0)
    sp0 = _aggregate_kernel(N, D, E, NPAD, SEC, SCH)(y0, s0, d0)
    y1 = pl.pallas_call(
        _tc_mid(N), out_shape=jax.ShapeDtypeStruct((N, D), f32))(
            sp0, degp, b0, W1)
    sp1 = _aggregate_kernel(N, D, E, NPAD, SEC, SCH)(y1, s1, d1)
    out = pl.pallas_call(
        _tc_post(N), out_shape=jax.ShapeDtypeStruct((N, D), f32))(
            sp1, degp, b1)
    return out


# speedup vs baseline: 1.1376x; 1.0207x over previous
import functools

import jax
import jax.numpy as jnp
from jax import lax
from jax.experimental import pallas as pl
from jax.experimental.pallas import tpu as pltpu
from jax.experimental.pallas import tpu_sc as plsc

NC = 2
NS = 16
NW = NC * NS
CH = 100

_MESH = plsc.VectorSubcoreMesh(core_axis_name="c", subcore_axis_name="s")


@functools.lru_cache(maxsize=None)
def _degree_kernel(E, NPAD, SEC, SCH):
    CP = NPAD // NS
    SP = ((CP + 15) // 16) * 16

    @functools.partial(
        pl.kernel,
        out_type=jax.ShapeDtypeStruct((NC * 4 * NPAD,), jnp.float32),
        mesh=_MESH,
        scratch_types=[
            pltpu.VMEM_SHARED((NPAD,), jnp.float32),
            pltpu.VMEM_SHARED((NPAD,), jnp.float32),
            pltpu.VMEM_SHARED((NPAD,), jnp.float32),
            pltpu.VMEM_SHARED((NPAD,), jnp.float32),
            pltpu.VMEM((SEC, SCH, CH), jnp.int32),
            pltpu.VMEM((SEC, SCH, CH), jnp.int32),
            pltpu.VMEM((SEC, SCH, CH), jnp.int32),
            pltpu.VMEM((SEC, SCH, CH), jnp.int32),
            pltpu.VMEM((CH,), jnp.float32),
            pltpu.VMEM((SP,), jnp.float32),
            pltpu.SemaphoreType.DMA,
        ],
    )
    def body(s0, d0, s1, d1, out, g0, g1, g2, g3,
             i0, i1, i2, i3, ones, stage, sem):
        c = lax.axis_index("c")
        s = lax.axis_index("s")
        wid = s * NC + c
        hists = (g0, g1, g2, g3)
        ibufs = (i0, i1, i2, i3)
        z16 = jnp.zeros((16,), jnp.float32)
        for j in range(SP // 16):
            stage[pl.ds(j * 16, 16)] = z16
        o16 = jnp.ones((16,), jnp.float32)
        for j in range(CH // 16):
            ones[pl.ds(j * 16, 16)] = o16
        for g in hists:
            pltpu.sync_copy(stage.at[pl.ds(0, CP)], g.at[pl.ds(s * CP, CP)])
        for arr, ib in zip((s0, d0, s1, d1), ibufs):
            pltpu.sync_copy(arr.at[wid], ib)
        plsc.subcore_barrier()
        for g, ib in zip(hists, ibufs):
            def osec(sec, _, g=g, ib=ib):
                def fire(k, _):
                    pltpu.async_copy(ones, g.at[ib.at[sec].at[k]],
                                     sem, add=True)
                    return _

                lax.fori_loop(0, SCH, fire, None)
                return _

            lax.fori_loop(0, SEC, osec, None)

        def drain(k, _):
            pltpu.make_async_copy(ones, g0.at[i0.at[0].at[0]], sem).wait()
            return _

        lax.fori_loop(0, 4 * SEC * SCH, drain, None)
        plsc.subcore_barrier()
        for a, g in enumerate(hists):
            pltpu.sync_copy(g.at[pl.ds(s * CP, CP)], stage.at[pl.ds(0, CP)])
            pltpu.sync_copy(stage.at[pl.ds(0, CP)],
                            out.at[pl.ds((c * 4 + a) * NPAD + s * CP, CP)])

    return body


@functools.lru_cache(maxsize=None)
def _aggregate_kernel(N, D, E, NR, SEC, SCH):
    assert SCH >= 5 and (SCH - 4) % 3 == 0
    NTRIP = (SCH - 4) // 3
    RP = NR // NS

    @functools.partial(
        pl.kernel,
        out_type=jax.ShapeDtypeStruct((NC, NR, D), jnp.float32),
        mesh=_MESH,
        scratch_types=[
            pltpu.VMEM_SHARED((NR, D), jnp.float32),
            pltpu.VMEM((SCH, CH), jnp.int32),
            pltpu.VMEM((SCH, CH), jnp.int32),
            pltpu.VMEM((CH, D), jnp.float32),
            pltpu.VMEM((CH, D), jnp.float32),
            pltpu.VMEM((CH, D), jnp.float32),
            pltpu.SemaphoreType.DMA,
            pltpu.SemaphoreType.DMA,
            pltpu.SemaphoreType.DMA,
            pltpu.SemaphoreType.DMA,
            pltpu.SemaphoreType.DMA,
            pltpu.SemaphoreType.DMA,
        ],
    )
    def body(y, src, dst, out, acc, sidx, didx,
             rows0, rows1, rows2, ga, gb, gc, sa, sb, sc_):
        c = lax.axis_index("c")
        s = lax.axis_index("s")
        wid = s * NC + c
        R = (rows0, rows1, rows2)
        G = (ga, gb, gc)
        S = (sa, sb, sc_)

        z16 = jnp.zeros((16,), jnp.float32)
        for r in range(8):
            for j in range(D // 16):
                rows0[r, pl.ds(j * 16, 16)] = z16

        def zfire(k, _):
            pltpu.async_copy(rows0.at[pl.ds(0, 8)],
                             acc.at[pl.ds(s * RP + k * 8, 8)], sa)
            return _

        lax.fori_loop(0, RP // 8, zfire, None)

        def zdrain(k, _):
            pltpu.make_async_copy(rows0.at[pl.ds(0, 8)],
                                  acc.at[pl.ds(s * RP, 8)], sa).wait()
            return _

        lax.fori_loop(0, RP // 8, zdrain, None)
        plsc.subcore_barrier()

        def gath(k, b):
            pltpu.async_copy(y.at[sidx.at[k]], R[b], G[b])

        def gwait(b):
            pltpu.make_async_copy(y.at[sidx.at[0]], R[b], G[b]).wait()

        def scat(k, b):
            pltpu.async_copy(R[b], acc.at[didx.at[k]], S[b], add=True)

        def swait(b):
            pltpu.make_async_copy(R[b], acc.at[didx.at[0]], S[b]).wait()

        def section(sec, _):
            pltpu.sync_copy(src.at[wid].at[sec], sidx)
            pltpu.sync_copy(dst.at[wid].at[sec], didx)
            gath(0, 0)
            gath(1, 1)
            gwait(0)
            scat(0, 0)
            gath(2, 2)
            gwait(1)
            scat(1, 1)
            def trip(j, _):
                base = 3 + 3 * j
                for t in range(3):
                    k = base + t
                    swait(t)
                    gath(k, t)
                    bp = (t + 2) % 3
                    gwait(bp)
                    scat(k - 1, bp)
                return _

            lax.fori_loop(0, NTRIP, trip, None)
            last = SCH - 1
            swait(last % 3)
            gath(last, last % 3)
            gwait((last + 2) % 3)
            scat(last - 1, (last + 2) % 3)
            gwait(last % 3)
            scat(last, last % 3)
            for b in range(3):
                swait(b)
            return _

        lax.fori_loop(0, SEC, section, None)
        plsc.subcore_barrier()

        WCH = CH - CH % 8
        nfull, rem = divmod(RP, WCH)
        sizes = [WCH] * nfull + ([rem] if rem else [])
        pend = [None, None, None]
        for m, sz in enumerate(sizes):
            b = m % 3
            if pend[b] is not None:
                po, psz = pend[b]
                pltpu.make_async_copy(
                    R[b].at[pl.ds(0, psz)],
                    out.at[c].at[pl.ds(s * RP + po, psz)], S[b]).wait()
            off = m * WCH
            pltpu.sync_copy(acc.at[pl.ds(s * RP + off, sz)],
                            R[b].at[pl.ds(0, sz)])
            pltpu.async_copy(R[b].at[pl.ds(0, sz)],
                             out.at[c].at[pl.ds(s * RP + off, sz)], S[b])
            pend[b] = (off, sz)
        for b in range(3):
            if pend[b] is not None:
                po, psz = pend[b]
                pltpu.make_async_copy(
                    R[b].at[pl.ds(0, psz)],
                    out.at[c].at[pl.ds(s * RP + po, psz)], S[b]).wait()

    return body


def _tc_pre(N):
    def body(x_ref, degp_ref, w_ref, y_ref):
        od = (degp_ref[0, 0, :] + degp_ref[1, 0, :])[:N]
        norm = lax.rsqrt(jnp.maximum(od, 1.0))
        h = x_ref[...] * norm[:, None]
        y_ref[...] = jnp.dot(h, w_ref[...], preferred_element_type=jnp.float32)
    return body


def _tc_mid(N):
    def body(sp_ref, degp_ref, b_ref, w_ref, y_ref):
        agg = sp_ref[0, :N] + sp_ref[1, :N]
        ind = (degp_ref[0, 1, :] + degp_ref[1, 1, :])[:N]
        od = (degp_ref[0, 2, :] + degp_ref[1, 2, :])[:N]
        h = jnp.maximum(agg * lax.rsqrt(jnp.maximum(ind, 1.0))[:, None]
                        + b_ref[...][None, :], 0.0)
        h = h * lax.rsqrt(jnp.maximum(od, 1.0))[:, None]
        y_ref[...] = jnp.dot(h, w_ref[...], preferred_element_type=jnp.float32)
    return body


def _tc_post(N):
    def body(sp_ref, degp_ref, b_ref, o_ref):
        agg = sp_ref[0, :N] + sp_ref[1, :N]
        ind = (degp_ref[0, 3, :] + degp_ref[1, 3, :])[:N]
        o_ref[...] = jnp.maximum(agg * lax.rsqrt(jnp.maximum(ind, 1.0))[:, None]
                                 + b_ref[...][None, :], 0.0)
    return body


def kernel(x, edge_index0, edge_index1, W0, b0, W1, b1):
    N, D = x.shape
    E = edge_index0.shape[1]
    NPAD = ((N + 127) // 128) * 128
    NCHUNK = E // NW // CH
    SCH = 25 if NCHUNK % 25 == 0 else NCHUNK
    SEC = NCHUNK // SCH

    s0 = edge_index0[0].reshape(NW, SEC, SCH, CH)
    d0 = edge_index0[1].reshape(NW, SEC, SCH, CH)
    s1 = edge_index1[0].reshape(NW, SEC, SCH, CH)
    d1 = edge_index1[1].reshape(NW, SEC, SCH, CH)

    degp = _degree_kernel(E, NPAD, SEC, SCH)(s0, d0, s1, d1)
    degp = degp.reshape(NC, 4, NPAD)

    f32 = jnp.float32
    y0 = pl.pallas_call(
        _tc_pre(N), out_shape=jax.ShapeDtypeStruct((N, D), f32))(x, degp, W0)
    sp0 = _aggregate_kernel(N, D, E, NPAD, SEC, SCH)(y0, s0, d0)
    y1 = pl.pallas_call(
        _tc_mid(N), out_shape=jax.ShapeDtypeStruct((N, D), f32))(
            sp0, degp, b0, W1)
    sp1 = _aggregate_kernel(N, D, E, NPAD, SEC, SCH)(y1, s1, d1)
    out = pl.pallas_call(
        _tc_post(N), out_shape=jax.ShapeDtypeStruct((N, D), f32))(
            sp1, degp, b1)
    return out
